# Initial kernel scaffold; baseline (speedup 1.0000x reference)
import jax
import jax.numpy as jnp
from jax.experimental import pallas as pl


def kernel(x, edge_index, edge_attr, params):
    raise NotImplementedError("write your pallas kernel here")


# speedup vs baseline: 1.8858x; 1.8858x over previous
import functools

import jax
import jax.numpy as jnp
from jax import lax
from jax.experimental import pallas as pl
from jax.experimental.pallas import tpu as pltpu
from jax.experimental.pallas import tpu_sc as plsc

_F32 = jnp.float32
_HIGH = jax.lax.Precision.HIGHEST


def _dot(a, b):
    return jnp.dot(a, b, preferred_element_type=_F32, precision=_HIGH)


def _edge_proj(edge_attr, wcat, bcat, n_layers, d):
    e_tot, k = edge_attr.shape
    blk = 4000

    def body(ea_ref, w_ref, b_ref, *outs):
        y = _dot(ea_ref[...], w_ref[...]) + b_ref[...]
        for i, o_ref in enumerate(outs):
            o_ref[...] = y[:, i * d:(i + 1) * d]

    return pl.pallas_call(
        body,
        grid=(e_tot // blk,),
        in_specs=[
            pl.BlockSpec((blk, k), lambda i: (i, 0)),
            pl.BlockSpec((k, n_layers * d), lambda i: (0, 0)),
            pl.BlockSpec((1, n_layers * d), lambda i: (0, 0)),
        ],
        out_specs=[pl.BlockSpec((blk, d), lambda i: (i, 0))] * n_layers,
        out_shape=[jax.ShapeDtypeStruct((e_tot, d), _F32)] * n_layers,
    )(edge_attr, wcat, bcat)


def _sc_message(h, e, src, dst, zeros):
    n, d = h.shape
    e_tot = src.shape[0]
    ch = 128
    n_chunks = e_tot // ch
    per_core = n_chunks // 2
    cp = (n // 16) // 8 * 8
    tail = n - 16 * cp
    mesh = plsc.VectorSubcoreMesh(core_axis_name="c", subcore_axis_name="s")

    @functools.partial(
        pl.kernel,
        out_type=jax.ShapeDtypeStruct((2, n, d), _F32),
        mesh=mesh,
        scratch_types=[
            pltpu.VMEM((ch,), jnp.int32),
            pltpu.VMEM((ch,), jnp.int32),
            pltpu.VMEM((ch, d), _F32),
            pltpu.VMEM((ch, d), _F32),
            pltpu.VMEM_SHARED((n, d), _F32),
            pltpu.SemaphoreType.DMA,
        ],
    )
    def k(h_hbm, e_hbm, src_hbm, dst_hbm, z_hbm, out_hbm,
          idx_s, idx_d, hrows, mrows, agg_sh, sem):
        c = lax.axis_index("c")
        s = lax.axis_index("s")

        @pl.when(s == 0)
        def _():
            pltpu.sync_copy(z_hbm, agg_sh)

        plsc.subcore_barrier()

        base = c * per_core
        n_my = (per_core - s + 15) // 16

        def chunk_body(kk, carry):
            g = base + s + 16 * kk
            eoff = g * ch
            pltpu.sync_copy(src_hbm.at[pl.ds(eoff, ch)], idx_s)
            pltpu.sync_copy(dst_hbm.at[pl.ds(eoff, ch)], idx_d)
            gth = pltpu.async_copy(h_hbm.at[idx_s], hrows, sem)
            pltpu.sync_copy(e_hbm.at[pl.ds(eoff, ch)], mrows)
            gth.wait()

            def row_body(r, carry2):
                for c16 in range(d // 16):
                    sl = pl.ds(c16 * 16, 16)
                    mrows[r, sl] = jnp.maximum(hrows[r, sl] + mrows[r, sl],
                                               0.0)
                return carry2

            lax.fori_loop(0, ch, row_body, 0)
            pltpu.sync_copy(mrows, agg_sh.at[idx_d], add=True)
            return carry

        lax.fori_loop(0, n_my, chunk_body, 0)
        plsc.subcore_barrier()

        off = s * cp
        pltpu.sync_copy(agg_sh.at[pl.ds(off, cp)],
                        out_hbm.at[c, pl.ds(off, cp)])

        @pl.when(s == 15)
        def _():
            pltpu.sync_copy(agg_sh.at[pl.ds(16 * cp, tail)],
                            out_hbm.at[c, pl.ds(16 * cp, tail)])

    return k(h, e, src, dst, zeros)


def _node_mlp(h, agg0, agg1, w1, b1, w2, b2, gamma, beta):
    n, d = h.shape
    hdim = w1.shape[1]
    blk = 1000

    def body(h_ref, a0_ref, a1_ref, w1_ref, b1_ref, w2_ref, b2_ref,
             g_ref, bt_ref, o_ref):
        hh = h_ref[...]
        a = hh + a0_ref[...] + a1_ref[...]
        z = jnp.maximum(_dot(a, w1_ref[...]) + b1_ref[...], 0.0)
        z = jnp.maximum(_dot(z, w2_ref[...]) + b2_ref[...], 0.0)
        t = z + hh
        mu = jnp.mean(t, axis=-1, keepdims=True)
        var = jnp.mean((t - mu) * (t - mu), axis=-1, keepdims=True)
        o_ref[...] = ((t - mu) * lax.rsqrt(var + 1e-5) * g_ref[...]
                      + bt_ref[...])

    full = lambda r, cdim: pl.BlockSpec((r, cdim), lambda i: (0, 0))
    return pl.pallas_call(
        body,
        grid=(n // blk,),
        in_specs=[
            pl.BlockSpec((blk, d), lambda i: (i, 0)),
            pl.BlockSpec((blk, d), lambda i: (i, 0)),
            pl.BlockSpec((blk, d), lambda i: (i, 0)),
            full(d, hdim), full(1, hdim), full(hdim, hdim), full(1, hdim),
            full(1, hdim), full(1, hdim),
        ],
        out_specs=pl.BlockSpec((blk, hdim), lambda i: (i, 0)),
        out_shape=jax.ShapeDtypeStruct((n, hdim), _F32),
    )(h, agg0, agg1, w1, b1, w2, b2, gamma, beta)


def _sc_gather2(h, src, dst):
    n, d = h.shape
    e_tot = src.shape[0]
    ch = 128
    n_chunks = e_tot // ch
    mesh = plsc.VectorSubcoreMesh(core_axis_name="c", subcore_axis_name="s")

    @functools.partial(
        pl.kernel,
        out_type=(jax.ShapeDtypeStruct((e_tot, d), _F32),
                  jax.ShapeDtypeStruct((e_tot, d), _F32)),
        mesh=mesh,
        scratch_types=[
            pltpu.VMEM((ch,), jnp.int32),
            pltpu.VMEM((ch,), jnp.int32),
            pltpu.VMEM((ch, d), _F32),
            pltpu.VMEM((ch, d), _F32),
            pltpu.SemaphoreType.DMA,
            pltpu.SemaphoreType.DMA,
        ],
    )
    def k(h_hbm, src_hbm, dst_hbm, os_hbm, od_hbm,
          idx_s, idx_d, rs, rd, sem_a, sem_b):
        c = lax.axis_index("c")
        s = lax.axis_index("s")
        wid = c * 16 + s
        n_my = (n_chunks - wid + 31) // 32

        def chunk_body(kk, carry):
            eoff = (wid + 32 * kk) * ch
            pltpu.sync_copy(src_hbm.at[pl.ds(eoff, ch)], idx_s)
            pltpu.sync_copy(dst_hbm.at[pl.ds(eoff, ch)], idx_d)
            ga = pltpu.async_copy(h_hbm.at[idx_s], rs, sem_a)
            gb = pltpu.async_copy(h_hbm.at[idx_d], rd, sem_b)
            ga.wait()
            gb.wait()
            pltpu.sync_copy(rs, os_hbm.at[pl.ds(eoff, ch)])
            pltpu.sync_copy(rd, od_hbm.at[pl.ds(eoff, ch)])
            return carry

        lax.fori_loop(0, n_my, chunk_body, 0)

    return k(h, src, dst)


def _decoder(hs, hd, ea, wa, wb, wc, wd, we, b1, w2, b2, w3, b3):
    e_tot, d = hs.shape
    k_e = ea.shape[1]
    dec = wa.shape[1]
    blk = 4000

    def body(hs_ref, hd_ref, ea_ref, wa_ref, wb_ref, wc_ref, wd_ref,
             we_ref, b1_ref, w2_ref, b2_ref, w3_ref, b3_ref, o_ref):
        a = hs_ref[...]
        b = hd_ref[...]
        q = (_dot(a, wa_ref[...]) + _dot(b, wb_ref[...])
             + _dot(jnp.abs(a - b), wc_ref[...])
             + _dot(a * b, wd_ref[...])
             + _dot(ea_ref[...], we_ref[...]) + b1_ref[...])
        q = jnp.maximum(q, 0.0)
        q = jnp.maximum(_dot(q, w2_ref[...]) + b2_ref[...], 0.0)
        o_ref[...] = _dot(q, w3_ref[...]) + b3_ref[...]

    full = lambda r, cdim: pl.BlockSpec((r, cdim), lambda i: (0, 0))
    out = pl.pallas_call(
        body,
        grid=(e_tot // blk,),
        in_specs=[
            pl.BlockSpec((blk, d), lambda i: (i, 0)),
            pl.BlockSpec((blk, d), lambda i: (i, 0)),
            pl.BlockSpec((blk, k_e), lambda i: (i, 0)),
            full(d, dec), full(d, dec), full(d, dec), full(d, dec),
            full(k_e, dec), full(1, dec),
            full(dec, dec // 2), full(1, dec // 2),
            full(dec // 2, 1), full(1, 1),
        ],
        out_specs=pl.BlockSpec((blk, 1), lambda i: (i, 0)),
        out_shape=jax.ShapeDtypeStruct((e_tot, 1), _F32),
    )(hs, hd, ea, wa, wb, wc, wd, we, b1, w2, b2, w3, b3)
    return out[:, 0]


def kernel(x, edge_index, edge_attr, params):
    n, d = x.shape
    src = edge_index[0]
    dst = edge_index[1]
    n_layers = 3

    wcat = jnp.concatenate([params[f"We_{i}"] for i in range(n_layers)],
                           axis=1)
    bcat = jnp.concatenate([params[f"be_{i}"] for i in range(n_layers)]
                           )[None, :]
    e_list = _edge_proj(edge_attr, wcat, bcat, n_layers, d)

    zeros = jnp.zeros((n, d), _F32)
    h = x
    for i in range(n_layers):
        agg = _sc_message(h, e_list[i], src, dst, zeros)
        h = _node_mlp(h, agg[0], agg[1],
                      params[f"W1_{i}"], params[f"b1_{i}"][None, :],
                      params[f"W2_{i}"], params[f"b2_{i}"][None, :],
                      params[f"gamma_{i}"][None, :],
                      params[f"beta_{i}"][None, :])

    hs, hd = _sc_gather2(h, src, dst)
    wd1 = params["Wd1"]
    return _decoder(
        hs, hd, edge_attr,
        wd1[0:d], wd1[d:2 * d], wd1[2 * d:3 * d], wd1[3 * d:4 * d],
        wd1[4 * d:], params["bd1"][None, :],
        params["Wd2"], params["bd2"][None, :],
        params["Wd3"], params["bd3"][None, :])


# speedup vs baseline: 3.3625x; 1.7830x over previous
import functools

import jax
import jax.numpy as jnp
from jax import lax
from jax.experimental import pallas as pl
from jax.experimental.pallas import tpu as pltpu
from jax.experimental.pallas import tpu_sc as plsc

_F32 = jnp.float32


def _dot(a, b):
    return jnp.dot(a, b, preferred_element_type=_F32)


def _dot_hi(a, b):
    return jnp.dot(a, b, preferred_element_type=_F32,
                   precision=jax.lax.Precision.HIGHEST)


_BF16 = jnp.bfloat16


def _split_w(w):
    hi = w.astype(_BF16)
    return hi, (w - hi.astype(_F32)).astype(_BF16)


def _dot3(a, w_hi, w_lo):
    a_hi = a.astype(_BF16)
    a_lo = (a - a_hi.astype(_F32)).astype(_BF16)
    return (_dot(a_hi, w_hi) + (_dot(a_hi, w_lo) + _dot(a_lo, w_hi)))


def _edge_proj(edge_attr, wcat, bcat, n_layers, d):
    e_tot, k = edge_attr.shape
    blk = 4000
    w_hi, w_lo = _split_w(wcat)

    def body(ea_ref, whi_ref, wlo_ref, b_ref, *outs):
        y = _dot3(ea_ref[...], whi_ref[...], wlo_ref[...]) + b_ref[...]
        for i, o_ref in enumerate(outs):
            o_ref[...] = y[:, i * d:(i + 1) * d]

    wspec = pl.BlockSpec((k, n_layers * d), lambda i: (0, 0))
    return pl.pallas_call(
        body,
        grid=(e_tot // blk,),
        in_specs=[
            pl.BlockSpec((blk, k), lambda i: (i, 0)),
            wspec, wspec,
            pl.BlockSpec((1, n_layers * d), lambda i: (0, 0)),
        ],
        out_specs=[pl.BlockSpec((blk, d), lambda i: (i, 0))] * n_layers,
        out_shape=[jax.ShapeDtypeStruct((e_tot, d), _F32)] * n_layers,
    )(edge_attr, w_hi, w_lo, bcat)


_NB = 3


def _sc_message(h, e, src, dst, zeros):
    n, d = h.shape
    e_tot = e.shape[0]
    ch = 80
    nb = 2
    n_chunks = e_tot // ch
    per_core = n_chunks // 2
    cp = (n // 16) // 8 * 8
    tail = n - 16 * cp
    mesh = plsc.VectorSubcoreMesh(core_axis_name="c", subcore_axis_name="s")

    @functools.partial(
        pl.kernel,
        out_type=jax.ShapeDtypeStruct((2, n, d), _F32),
        mesh=mesh,
        scratch_types=(
            [pltpu.VMEM((ch,), jnp.int32)] * (2 * nb)
            + [pltpu.VMEM((ch, d), _F32)] * (2 * nb)
            + [pltpu.VMEM_SHARED((n, d), _F32)]
            + [pltpu.SemaphoreType.DMA] * (4 * nb)
        ),
    )
    def k(h_hbm, e_hbm, src_hbm, dst_hbm, z_hbm, out_hbm, *scr):
        idxs = scr[0:nb]
        idxd = scr[nb:2 * nb]
        hro = scr[2 * nb:3 * nb]
        mro = scr[3 * nb:4 * nb]
        agg_sh = scr[4 * nb]
        isem = scr[4 * nb + 1:4 * nb + 1 + nb]
        gsem = scr[4 * nb + 1 + nb:4 * nb + 1 + 2 * nb]
        esem = scr[4 * nb + 1 + 2 * nb:4 * nb + 1 + 3 * nb]
        ssem = scr[4 * nb + 1 + 3 * nb:4 * nb + 1 + 4 * nb]

        c = lax.axis_index("c")
        s = lax.axis_index("s")

        @pl.when(s == 0)
        def _():
            pltpu.sync_copy(z_hbm, agg_sh)

        plsc.subcore_barrier()

        base = c * per_core
        n_my = (per_core - s + 15) // 16
        n_groups = (n_my + nb - 1) // nb

        def eoff_of(t):
            return (base + s + 16 * t) * ch

        def group_body(kk, carry):
            for b in range(nb):
                t = nb * kk + b

                @pl.when((t < n_my) & (kk > 0))
                def _(b=b):
                    pltpu.make_async_copy(
                        mro[b], agg_sh.at[idxd[b]], ssem[b]).wait()

                @pl.when(t < n_my)
                def _(b=b, t=t):
                    eo = eoff_of(t)
                    pltpu.async_copy(src_hbm.at[pl.ds(eo, ch)],
                                     idxs[b], isem[b])
                    pltpu.async_copy(dst_hbm.at[pl.ds(eo, ch)],
                                     idxd[b], isem[b])

            for b in range(nb):
                t = nb * kk + b

                @pl.when(t < n_my)
                def _(b=b, t=t):
                    eo = eoff_of(t)
                    pltpu.make_async_copy(src_hbm.at[pl.ds(eo, ch)],
                                          idxs[b], isem[b]).wait()
                    pltpu.make_async_copy(dst_hbm.at[pl.ds(eo, ch)],
                                          idxd[b], isem[b]).wait()
                    pltpu.async_copy(h_hbm.at[idxs[b]], hro[b], gsem[b])
                    pltpu.async_copy(e_hbm.at[pl.ds(eo, ch)],
                                     mro[b], esem[b])

            for b in range(nb):
                t = nb * kk + b

                @pl.when(t < n_my)
                def _(b=b, t=t):
                    pltpu.make_async_copy(h_hbm.at[idxs[b]], hro[b],
                                          gsem[b]).wait()
                    pltpu.make_async_copy(e_hbm.at[pl.ds(eoff_of(t), ch)],
                                          mro[b], esem[b]).wait()

                    def row_body(r, carry2):
                        for c16 in range(d // 16):
                            sl = pl.ds(c16 * 16, 16)
                            mro[b][r, sl] = jnp.maximum(
                                hro[b][r, sl] + mro[b][r, sl], 0.0)
                        return carry2

                    lax.fori_loop(0, ch, row_body, 0)
                    pltpu.async_copy(mro[b], agg_sh.at[idxd[b]],
                                     ssem[b], add=True)

            return carry

        lax.fori_loop(0, n_groups, group_body, 0)
        for b in range(nb):
            pltpu.make_async_copy(mro[b], agg_sh.at[idxd[b]],
                                  ssem[b]).wait()
        plsc.subcore_barrier()

        off = s * cp
        pltpu.sync_copy(agg_sh.at[pl.ds(off, cp)],
                        out_hbm.at[c, pl.ds(off, cp)])

        @pl.when(s == 15)
        def _():
            pltpu.sync_copy(agg_sh.at[pl.ds(16 * cp, tail)],
                            out_hbm.at[c, pl.ds(16 * cp, tail)])

    return k(h, e, src, dst, zeros)


def _node_mlp(h, agg, w1, b1, w2, b2, gamma, beta):
    n, d = h.shape
    hdim = w1.shape[1]
    blk = 1000
    w1_hi, w1_lo = _split_w(w1)
    w2_hi, w2_lo = _split_w(w2)

    def body(h_ref, a_ref, w1h_ref, w1l_ref, b1_ref, w2h_ref, w2l_ref,
             b2_ref, g_ref, bt_ref, o_ref):
        hh = h_ref[...]
        a = hh + a_ref[0] + a_ref[1]
        z = jnp.maximum(_dot3(a, w1h_ref[...], w1l_ref[...]) + b1_ref[...],
                        0.0)
        z = jnp.maximum(_dot3(z, w2h_ref[...], w2l_ref[...]) + b2_ref[...],
                        0.0)
        t = z + hh
        mu = jnp.mean(t, axis=-1, keepdims=True)
        var = jnp.mean((t - mu) * (t - mu), axis=-1, keepdims=True)
        o_ref[...] = ((t - mu) * lax.rsqrt(var + 1e-5) * g_ref[...]
                      + bt_ref[...])

    full = lambda r, cdim: pl.BlockSpec((r, cdim), lambda i: (0, 0))
    return pl.pallas_call(
        body,
        grid=(n // blk,),
        in_specs=[
            pl.BlockSpec((blk, d), lambda i: (i, 0)),
            pl.BlockSpec((2, blk, d), lambda i: (0, i, 0)),
            full(d, hdim), full(d, hdim), full(1, hdim),
            full(hdim, hdim), full(hdim, hdim), full(1, hdim),
            full(1, hdim), full(1, hdim),
        ],
        out_specs=pl.BlockSpec((blk, hdim), lambda i: (i, 0)),
        out_shape=jax.ShapeDtypeStruct((n, hdim), _F32),
    )(h, agg, w1_hi, w1_lo, b1, w2_hi, w2_lo, b2, gamma, beta)


def _sc_gather2(h, ei, lo, cnt):
    n, d = h.shape
    ch = 128
    n_chunks = cnt // ch
    mesh = plsc.VectorSubcoreMesh(core_axis_name="c", subcore_axis_name="s")

    @functools.partial(
        pl.kernel,
        out_type=(jax.ShapeDtypeStruct((cnt, d), _F32),
                  jax.ShapeDtypeStruct((cnt, d), _F32)),
        mesh=mesh,
        scratch_types=(
            [pltpu.VMEM((2, ch), jnp.int32)] * _NB
            + [pltpu.VMEM((ch, d), _F32)] * (2 * _NB)
            + [pltpu.SemaphoreType.DMA] * (5 * _NB)
        ),
    )
    def k(h_hbm, ei_hbm, os_hbm, od_hbm, *scr):
        idx = scr[0:_NB]
        rs = scr[_NB:2 * _NB]
        rd = scr[2 * _NB:3 * _NB]
        sems = scr[3 * _NB:]
        isem = sems[0:_NB]
        ga_sem = sems[_NB:2 * _NB]
        gb_sem = sems[2 * _NB:3 * _NB]
        wa_sem = sems[3 * _NB:4 * _NB]
        wb_sem = sems[4 * _NB:5 * _NB]

        c = lax.axis_index("c")
        s = lax.axis_index("s")
        wid = c * 16 + s
        n_my = (n_chunks - wid + 31) // 32
        n_groups = (n_my + _NB - 1) // _NB

        def off_of(t):
            return (wid + 32 * t) * ch

        def group_body(kk, carry):
            for b in range(_NB):
                t = _NB * kk + b

                @pl.when((t < n_my) & (kk > 0))
                def _(b=b, t=t):
                    po = off_of(t - _NB)
                    pltpu.make_async_copy(
                        rs[b], os_hbm.at[pl.ds(po, ch)], wa_sem[b]).wait()
                    pltpu.make_async_copy(
                        rd[b], od_hbm.at[pl.ds(po, ch)], wb_sem[b]).wait()

                @pl.when(t < n_my)
                def _(b=b, t=t):
                    pltpu.async_copy(
                        ei_hbm.at[:, pl.ds(lo + off_of(t), ch)],
                        idx[b], isem[b])

            for b in range(_NB):
                t = _NB * kk + b

                @pl.when(t < n_my)
                def _(b=b, t=t):
                    pltpu.make_async_copy(
                        ei_hbm.at[:, pl.ds(lo + off_of(t), ch)], idx[b],
                        isem[b]).wait()
                    pltpu.async_copy(h_hbm.at[idx[b].at[0]], rs[b],
                                     ga_sem[b])
                    pltpu.async_copy(h_hbm.at[idx[b].at[1]], rd[b],
                                     gb_sem[b])

            for b in range(_NB):
                t = _NB * kk + b

                @pl.when(t < n_my)
                def _(b=b, t=t):
                    pltpu.make_async_copy(h_hbm.at[idx[b].at[0]], rs[b],
                                          ga_sem[b]).wait()
                    pltpu.make_async_copy(h_hbm.at[idx[b].at[1]], rd[b],
                                          gb_sem[b]).wait()
                    pltpu.async_copy(rs[b], os_hbm.at[pl.ds(off_of(t), ch)],
                                     wa_sem[b])
                    pltpu.async_copy(rd[b], od_hbm.at[pl.ds(off_of(t), ch)],
                                     wb_sem[b])

            return carry

        lax.fori_loop(0, n_groups, group_body, 0)
        for b in range(_NB):
            @pl.when(jnp.int32(b) < n_my)
            def _(b=b):
                po = off_of(jnp.maximum(n_my - _NB, 0))
                pltpu.make_async_copy(
                    rs[b], os_hbm.at[pl.ds(po, ch)], wa_sem[b]).wait()
                pltpu.make_async_copy(
                    rd[b], od_hbm.at[pl.ds(po, ch)], wb_sem[b]).wait()

    return k(h, ei)


def _decoder(hs, hd, ea, lo, wa, wb, wc, wd, we, b1, w2, b2, w3, b3):
    e_tot, d = hs.shape
    k_e = ea.shape[1]
    dec = wa.shape[1]
    blk = 4000
    off = lo // blk
    splits = [_split_w(w) for w in (wa, wb, wc, wd, we, w2, w3)]
    w_his = [s[0] for s in splits]
    w_los = [s[1] for s in splits]

    def body(hs_ref, hd_ref, ea_ref,
             wah, wbh, wch, wdh, weh, w2h, w3h,
             wal, wbl, wcl, wdl, wel, w2l, w3l,
             b1_ref, b2_ref, b3_ref, o_ref):
        a = hs_ref[...]
        b = hd_ref[...]
        q = (_dot3(a, wah[...], wal[...]) + _dot3(b, wbh[...], wbl[...])
             + _dot3(jnp.abs(a - b), wch[...], wcl[...])
             + _dot3(a * b, wdh[...], wdl[...])
             + _dot3(ea_ref[...], weh[...], wel[...]) + b1_ref[...])
        q = jnp.maximum(q, 0.0)
        q = jnp.maximum(_dot3(q, w2h[...], w2l[...]) + b2_ref[...], 0.0)
        o_ref[...] = _dot3(q, w3h[...], w3l[...]) + b3_ref[...]

    full = lambda r, cdim: pl.BlockSpec((r, cdim), lambda i: (0, 0))
    wspecs = [full(d, dec)] * 4 + [full(k_e, dec),
                                   full(dec, dec // 2), full(dec // 2, 1)]
    out = pl.pallas_call(
        body,
        grid=(e_tot // blk,),
        in_specs=(
            [pl.BlockSpec((blk, d), lambda i: (i, 0)),
             pl.BlockSpec((blk, d), lambda i: (i, 0)),
             pl.BlockSpec((blk, k_e), lambda i: (i + off, 0))]
            + wspecs + wspecs
            + [full(1, dec), full(1, dec // 2), full(1, 1)]
        ),
        out_specs=pl.BlockSpec((blk, 1), lambda i: (i, 0)),
        out_shape=jax.ShapeDtypeStruct((e_tot, 1), _F32),
    )(hs, hd, ea, *w_his, *w_los, b1, b2, b3)
    return out[:, 0]


def kernel(x, edge_index, edge_attr, params):
    n, d = x.shape
    n_layers = 3

    (e0,) = _edge_proj(edge_attr, params["We_0"],
                       params["be_0"][None, :], 1, d)
    wcat = jnp.concatenate([params["We_1"], params["We_2"]], axis=1)
    bcat = jnp.concatenate([params["be_1"], params["be_2"]])[None, :]
    e1, e2 = _edge_proj(edge_attr, wcat, bcat, 2, d)
    e_list = [e0, e1, e2]

    zeros = jnp.zeros((n, d), _F32)
    src = edge_index[0]
    dst = edge_index[1]
    h = x
    for i in range(n_layers):
        agg = _sc_message(h, e_list[i], src, dst, zeros)
        h = _node_mlp(h, agg,
                      params[f"W1_{i}"], params[f"b1_{i}"][None, :],
                      params[f"W2_{i}"], params[f"b2_{i}"][None, :],
                      params[f"gamma_{i}"][None, :],
                      params[f"beta_{i}"][None, :])

    wd1 = params["Wd1"]
    half = edge_index.shape[1] // 2
    outs = []
    for lo in (0, half):
        hs, hd = _sc_gather2(h, edge_index, lo, half)
        outs.append(_decoder(
            hs, hd, edge_attr, lo,
            wd1[0:d], wd1[d:2 * d], wd1[2 * d:3 * d], wd1[3 * d:4 * d],
            wd1[4 * d:], params["bd1"][None, :],
            params["Wd2"], params["bd2"][None, :],
            params["Wd3"], params["bd3"][None, :]))
    return jnp.concatenate(outs)
